# baseline (speedup 1.0000x reference)
import functools
import math

import jax
import jax.numpy as jnp
from jax import lax
from jax.experimental import pallas as pl
from jax.experimental.pallas import tpu as pltpu
from jax.experimental.pallas import tpu_sc as plsc

_BN = 1000
_CH = 40
_NC = 2
_NS = 16
_ZR = 40


def _leaky(t):
    return jnp.where(t >= 0, t, 0.01 * t)


def _rsqrt(u):
    r = lax.rsqrt(u)
    return r * (1.5 - 0.5 * u * r * r)


def _proj_stats_body(x_ref, w_ref, b_ref, h_ref, s1_ref, s2_ref):
    i = pl.program_id(0)
    h = _leaky(jnp.dot(x_ref[...], w_ref[...],
                       preferred_element_type=jnp.float32) + b_ref[...])
    h_ref[...] = h

    @pl.when(i == 0)
    def _():
        s1_ref[...] = jnp.zeros_like(s1_ref)
        s2_ref[...] = jnp.zeros_like(s2_ref)

    s1_ref[...] += jnp.sum(h, axis=0, keepdims=True)
    s2_ref[...] += jnp.sum(h * h, axis=0, keepdims=True)


def _norm_qkv_body(hp_ref, s1_ref, s2_ref, g_ref, b_ref, qw_ref, kw_ref,
                   vw_ref, h_ref, q_ref, kv_ref, *, n_nodes):
    mu = s1_ref[...] / n_nodes
    var = s2_ref[...] / n_nodes - mu * mu
    a = g_ref[...] * _rsqrt(var + 1e-5)
    b = b_ref[...] - mu * a
    h = hp_ref[...] * a + b
    h_ref[...] = h
    d = h.shape[-1]
    q_ref[...] = jnp.dot(h, qw_ref[...], preferred_element_type=jnp.float32)
    kv_ref[:, :d] = jnp.dot(h, kw_ref[...], preferred_element_type=jnp.float32)
    kv_ref[:, d:] = jnp.dot(h, vw_ref[...], preferred_element_type=jnp.float32)


def _post_layer(h, parts, ow, ob, lng, lnb):
    agg = parts[0] + parts[1]
    out = jnp.dot(agg, ow, preferred_element_type=jnp.float32) + ob
    hr = h + out
    m = jnp.mean(hr, axis=-1, keepdims=True)
    c = hr - m
    vv = jnp.mean(c * c, axis=-1, keepdims=True)
    return c * _rsqrt(vv + 1e-5) * lng + lnb


def _post_qkv_body(h_ref, p_ref, ow_ref, ob_ref, lng_ref, lnb_ref,
                   qw_ref, kw_ref, vw_ref,
                   h_out_ref, q_ref, kv_ref):
    hn = _post_layer(h_ref[...], p_ref[...], ow_ref[...], ob_ref[...],
                     lng_ref[...], lnb_ref[...])
    h_out_ref[...] = hn
    d = hn.shape[-1]
    q_ref[...] = jnp.dot(hn, qw_ref[...], preferred_element_type=jnp.float32)
    kv_ref[:, :d] = jnp.dot(hn, kw_ref[...], preferred_element_type=jnp.float32)
    kv_ref[:, d:] = jnp.dot(hn, vw_ref[...], preferred_element_type=jnp.float32)


def _post_pool_body(h_ref, p_ref, ow_ref, ob_ref, lng_ref, lnb_ref, s_ref):
    i = pl.program_id(0)
    hn = _post_layer(h_ref[...], p_ref[...], ow_ref[...], ob_ref[...],
                     lng_ref[...], lnb_ref[...])

    @pl.when(i == 0)
    def _():
        s_ref[...] = jnp.zeros_like(s_ref)

    s_ref[...] += jnp.sum(hn, axis=0, keepdims=True)


def _mlp_body(s_ref, w1a_ref, w1b_ref, b1_ref, w2_ref, b2_ref, y_ref, *,
              n_nodes):
    s = s_ref[...]
    pooled = jnp.dot(s / n_nodes, w1a_ref[...],
                     preferred_element_type=jnp.float32)
    pooled += jnp.dot(s, w1b_ref[...], preferred_element_type=jnp.float32)
    z = _leaky(pooled + b1_ref[...])
    y_ref[...] = jnp.dot(z, w2_ref[...],
                         preferred_element_type=jnp.float32) + b2_ref[...]


def _edge_sc_kernel(n_nodes, n_edges, d):
    n_chunks = n_edges // _CH
    nw = _NC * _NS
    n_rchunks = n_nodes // _ZR
    scale = 1.0 / math.sqrt(d)
    mesh = plsc.VectorSubcoreMesh(core_axis_name="c", subcore_axis_name="s",
                                  num_cores=_NC, num_subcores=_NS)

    chunks_per_w = n_chunks // nw
    assert chunks_per_w * nw == n_chunks and chunks_per_w % 2 == 0

    @functools.partial(
        pl.kernel,
        out_type=jax.ShapeDtypeStruct((_NC, n_nodes, d), jnp.float32),
        mesh=mesh,
        compiler_params=pltpu.CompilerParams(needs_layout_passes=False),
        scratch_types=[
            [pltpu.VMEM((_CH,), jnp.int32)] * 2,
            [pltpu.VMEM((_CH,), jnp.int32)] * 2,
            [pltpu.VMEM((_CH, d), jnp.float32)] * 2,
            [pltpu.VMEM((_CH, 2 * d), jnp.float32)] * 2,
            pltpu.VMEM((_CH, d), jnp.float32),
            pltpu.VMEM_SHARED((n_nodes, d), jnp.float32),
            [pltpu.SemaphoreType.DMA] * 2,
            [pltpu.SemaphoreType.DMA] * 2,
        ],
    )
    def edge_kernel(qn_hbm, kvn_hbm, src_hbm, dst_hbm, out_hbm,
                    sidx, didx, qv, kvv, msg, aggsh, semi, semg):
        cid = lax.axis_index("c")
        sid = lax.axis_index("s")
        wid = cid * _NS + sid

        def zrow(r, carry):
            for j in range(d // 16):
                msg[r, pl.ds(j * 16, 16)] = jnp.zeros((16,), jnp.float32)
            return carry

        lax.fori_loop(0, _ZR, zrow, 0)
        my_rchunks = (n_rchunks - sid + _NS - 1) // _NS

        def zchunk(t, carry):
            r0 = pl.multiple_of((sid + t * _NS) * _ZR, 8)
            pltpu.sync_copy(msg.at[pl.ds(0, _ZR)], aggsh.at[pl.ds(r0, _ZR)])
            return carry

        lax.fori_loop(0, my_rchunks, zchunk, 0)
        plsc.subcore_barrier()

        def ebase(c):
            return pl.multiple_of((wid + c * nw) * _CH, 8)

        def issue_idx(c, b):
            pltpu.async_copy(src_hbm.at[pl.ds(ebase(c), _CH)], sidx[b],
                             semi[b])
            pltpu.async_copy(dst_hbm.at[pl.ds(ebase(c), _CH)], didx[b],
                             semi[b])

        def wait_idx(b):
            pltpu.make_async_copy(src_hbm.at[pl.ds(0, _CH)], sidx[b],
                                  semi[b]).wait()
            pltpu.make_async_copy(dst_hbm.at[pl.ds(0, _CH)], didx[b],
                                  semi[b]).wait()

        def issue_gather(b):
            pltpu.async_copy(qn_hbm.at[didx[b]], qv[b], semg[b])
            pltpu.async_copy(kvn_hbm.at[sidx[b]], kvv[b], semg[b])

        def wait_gather(b):
            pltpu.make_async_copy(qn_hbm.at[didx[b]], qv[b], semg[b]).wait()
            pltpu.make_async_copy(kvn_hbm.at[sidx[b]], kvv[b],
                                  semg[b]).wait()

        def compute(b):
            qb, kvb = qv[b], kvv[b]

            def edge_body(e, c2):
                acc = qb[e, pl.ds(0, 16)] * kvb[e, pl.ds(0, 16)]
                for j in range(1, d // 16):
                    acc += qb[e, pl.ds(j * 16, 16)] * kvb[e, pl.ds(j * 16, 16)]
                logit = jnp.sum(acc) * scale
                lv = jnp.zeros((16,), jnp.float32) + logit
                alpha = 1.0 / (1.0 + jnp.exp(-lv))
                for j in range(d // 16):
                    msg[e, pl.ds(j * 16, 16)] = (
                        alpha * kvb[e, pl.ds(d + j * 16, 16)])
                return c2

            lax.fori_loop(0, _CH, edge_body, 0)

        for b in range(2):
            issue_idx(b, b)
            wait_idx(b)
            issue_gather(b)

        def pair_body(i, carry):
            for b in range(2):
                c = 2 * i + b
                wait_gather(b)
                compute(b)
                pltpu.sync_copy(msg, aggsh.at[didx[b]], add=True)

                @pl.when(c + 2 < chunks_per_w)
                def _():
                    issue_idx(c + 2, b)
                    wait_idx(b)
                    issue_gather(b)

            return carry

        lax.fori_loop(0, chunks_per_w // 2, pair_body, 0)
        plsc.subcore_barrier()

        def ochunk(t, carry):
            r0 = pl.multiple_of((sid + t * _NS) * _ZR, 8)
            pltpu.sync_copy(aggsh.at[pl.ds(r0, _ZR)], msg.at[pl.ds(0, _ZR)])
            pltpu.sync_copy(msg.at[pl.ds(0, _ZR)],
                            out_hbm.at[cid, pl.ds(r0, _ZR)])
            return carry

        lax.fori_loop(0, my_rchunks, ochunk, 0)

    return edge_kernel


def kernel(x, edge_index, inW, inb, bn_g, bn_b, qW, kW, vW, oW, ob, ln_g,
           ln_b, W1, b1, W2, b2):
    n, d = x.shape
    e = edge_index.shape[1]
    n_layers = qW.shape[0]
    grid = n // _BN

    blk = lambda: pl.BlockSpec((_BN, d), lambda i: (i, 0))
    blk2 = lambda: pl.BlockSpec((_BN, 2 * d), lambda i: (i, 0))
    row = lambda r=1: pl.BlockSpec((r, d), lambda i: (0, 0))
    wspec = lambda: pl.BlockSpec((d, d), lambda i: (0, 0))

    h_pre, s1, s2 = pl.pallas_call(
        _proj_stats_body,
        grid=(grid,),
        in_specs=[blk(), wspec(), row()],
        out_specs=[blk(), row(), row()],
        out_shape=[
            jax.ShapeDtypeStruct((n, d), jnp.float32),
            jax.ShapeDtypeStruct((1, d), jnp.float32),
            jax.ShapeDtypeStruct((1, d), jnp.float32),
        ],
    )(x, inW, inb.reshape(1, d))

    h, qn, kvn = pl.pallas_call(
        functools.partial(_norm_qkv_body, n_nodes=float(n)),
        grid=(grid,),
        in_specs=[blk(), row(), row(), row(), row(), wspec(), wspec(),
                  wspec()],
        out_specs=[blk(), blk(), blk2()],
        out_shape=[
            jax.ShapeDtypeStruct((n, d), jnp.float32),
            jax.ShapeDtypeStruct((n, d), jnp.float32),
            jax.ShapeDtypeStruct((n, 2 * d), jnp.float32),
        ],
    )(h_pre, s1, s2, bn_g.reshape(1, d), bn_b.reshape(1, d),
      qW[0], kW[0], vW[0])

    edge_fn = _edge_sc_kernel(n, e, d)
    pspec = pl.BlockSpec((_NC, _BN, d), lambda i: (0, i, 0))

    for i in range(n_layers):
        parts = edge_fn(qn, kvn, edge_index[0], edge_index[1])
        if i + 1 < n_layers:
            h, qn, kvn = pl.pallas_call(
                _post_qkv_body,
                grid=(grid,),
                in_specs=[blk(), pspec, wspec(), row(), row(), row(),
                          wspec(), wspec(), wspec()],
                out_specs=[blk(), blk(), blk2()],
                out_shape=[
                    jax.ShapeDtypeStruct((n, d), jnp.float32),
                    jax.ShapeDtypeStruct((n, d), jnp.float32),
                    jax.ShapeDtypeStruct((n, 2 * d), jnp.float32),
                ],
            )(h, parts, oW[i], ob[i].reshape(1, d), ln_g[i].reshape(1, d),
              ln_b[i].reshape(1, d), qW[i + 1], kW[i + 1], vW[i + 1])
        else:
            s = pl.pallas_call(
                _post_pool_body,
                grid=(grid,),
                in_specs=[blk(), pspec, wspec(), row(), row(), row()],
                out_specs=row(),
                out_shape=jax.ShapeDtypeStruct((1, d), jnp.float32),
            )(h, parts, oW[i], ob[i].reshape(1, d), ln_g[i].reshape(1, d),
              ln_b[i].reshape(1, d))

    y = pl.pallas_call(
        functools.partial(_mlp_body, n_nodes=float(n)),
        grid=(1,),
        in_specs=[row(), wspec(), wspec(), row(),
                  pl.BlockSpec((d, 1), lambda i: (0, 0)),
                  pl.BlockSpec((1, 1), lambda i: (0, 0))],
        out_specs=pl.BlockSpec((1, 1), lambda i: (0, 0)),
        out_shape=jax.ShapeDtypeStruct((1, 1), jnp.float32),
    )(s, W1[:d], W1[d:], b1.reshape(1, d), W2, b2.reshape(1, 1))

    return y.reshape(())


# speedup vs baseline: 2.7196x; 2.7196x over previous
import functools
import math

import jax
import jax.numpy as jnp
from jax import lax
from jax.experimental import pallas as pl
from jax.experimental.pallas import tpu as pltpu
from jax.experimental.pallas import tpu_sc as plsc

_BN = 1000
_CH = 40
_NC = 2
_NS = 16
_ZR = 40


def _leaky(t):
    return jnp.where(t >= 0, t, 0.01 * t)


def _rsqrt(u):
    r = lax.rsqrt(u)
    return r * (1.5 - 0.5 * u * r * r)


def _proj_stats_body(x_ref, w_ref, b_ref, h_ref, s1_ref, s2_ref):
    i = pl.program_id(0)
    h = _leaky(jnp.dot(x_ref[...], w_ref[...],
                       preferred_element_type=jnp.float32) + b_ref[...])
    h_ref[...] = h

    @pl.when(i == 0)
    def _():
        s1_ref[...] = jnp.zeros_like(s1_ref)
        s2_ref[...] = jnp.zeros_like(s2_ref)

    s1_ref[...] += jnp.sum(h, axis=0, keepdims=True)
    s2_ref[...] += jnp.sum(h * h, axis=0, keepdims=True)


def _norm_qkv_body(hp_ref, s1_ref, s2_ref, g_ref, b_ref, qw_ref, kw_ref,
                   vw_ref, h_ref, q_ref, k_ref, v_ref, *, n_nodes):
    mu = s1_ref[...] / n_nodes
    var = s2_ref[...] / n_nodes - mu * mu
    a = g_ref[...] * _rsqrt(var + 1e-5)
    b = b_ref[...] - mu * a
    h = hp_ref[...] * a + b
    h_ref[...] = h
    q_ref[...] = jnp.dot(h, qw_ref[...], preferred_element_type=jnp.float32)
    k_ref[...] = jnp.dot(h, kw_ref[...], preferred_element_type=jnp.float32)
    v_ref[...] = jnp.dot(h, vw_ref[...], preferred_element_type=jnp.float32)


def _post_layer(h, parts, ow, ob, lng, lnb):
    agg = parts[0] + parts[1]
    out = jnp.dot(agg, ow, preferred_element_type=jnp.float32) + ob
    hr = h + out
    m = jnp.mean(hr, axis=-1, keepdims=True)
    c = hr - m
    vv = jnp.mean(c * c, axis=-1, keepdims=True)
    return c * _rsqrt(vv + 1e-5) * lng + lnb


def _post_qkv_body(h_ref, p_ref, ow_ref, ob_ref, lng_ref, lnb_ref,
                   qw_ref, kw_ref, vw_ref,
                   h_out_ref, q_ref, k_ref, v_ref):
    hn = _post_layer(h_ref[...], p_ref[...], ow_ref[...], ob_ref[...],
                     lng_ref[...], lnb_ref[...])
    h_out_ref[...] = hn
    q_ref[...] = jnp.dot(hn, qw_ref[...], preferred_element_type=jnp.float32)
    k_ref[...] = jnp.dot(hn, kw_ref[...], preferred_element_type=jnp.float32)
    v_ref[...] = jnp.dot(hn, vw_ref[...], preferred_element_type=jnp.float32)


def _post_pool_body(h_ref, p_ref, ow_ref, ob_ref, lng_ref, lnb_ref, s_ref):
    i = pl.program_id(0)
    hn = _post_layer(h_ref[...], p_ref[...], ow_ref[...], ob_ref[...],
                     lng_ref[...], lnb_ref[...])

    @pl.when(i == 0)
    def _():
        s_ref[...] = jnp.zeros_like(s_ref)

    s_ref[...] += jnp.sum(hn, axis=0, keepdims=True)


def _mlp_body(s_ref, w1a_ref, w1b_ref, b1_ref, w2_ref, b2_ref, y_ref, *,
              n_nodes):
    s = s_ref[...]
    pooled = jnp.dot(s / n_nodes, w1a_ref[...],
                     preferred_element_type=jnp.float32)
    pooled += jnp.dot(s, w1b_ref[...], preferred_element_type=jnp.float32)
    z = _leaky(pooled + b1_ref[...])
    y_ref[...] = jnp.dot(z, w2_ref[...],
                         preferred_element_type=jnp.float32) + b2_ref[...]


def _edge_sc_kernel(n_nodes, n_edges, d):
    n_chunks = n_edges // _CH
    nw = _NC * _NS
    n_rchunks = n_nodes // _ZR
    scale = 1.0 / math.sqrt(d)
    mesh = plsc.VectorSubcoreMesh(core_axis_name="c", subcore_axis_name="s",
                                  num_cores=_NC, num_subcores=_NS)

    chunks_per_w = n_chunks // nw
    assert chunks_per_w * nw == n_chunks and chunks_per_w % 2 == 0

    @functools.partial(
        pl.kernel,
        out_type=jax.ShapeDtypeStruct((_NC, n_nodes, d), jnp.float32),
        mesh=mesh,
        compiler_params=pltpu.CompilerParams(needs_layout_passes=False),
        scratch_types=[
            [pltpu.VMEM((_CH,), jnp.int32)] * 2,
            [pltpu.VMEM((_CH,), jnp.int32)] * 2,
            [pltpu.VMEM((_CH, d), jnp.float32)] * 2,
            [pltpu.VMEM((_CH, d), jnp.float32)] * 2,
            [pltpu.VMEM((_CH, d), jnp.float32)] * 2,
            pltpu.VMEM_SHARED((n_nodes, d), jnp.float32),
            [pltpu.SemaphoreType.DMA] * 2,
            [pltpu.SemaphoreType.DMA] * 2,
        ],
    )
    def edge_kernel(qn_hbm, kn_hbm, vn_hbm, src_hbm, dst_hbm, out_hbm,
                    sidx, didx, qv, kv, vv, aggsh, semi, semg):
        cid = lax.axis_index("c")
        sid = lax.axis_index("s")
        wid = cid * _NS + sid

        def zrow(r, carry):
            for j in range(d // 16):
                qv[0][r, pl.ds(j * 16, 16)] = jnp.zeros((16,), jnp.float32)
            return carry

        lax.fori_loop(0, _ZR, zrow, 0)
        my_rchunks = (n_rchunks - sid + _NS - 1) // _NS

        def zchunk(t, carry):
            r0 = pl.multiple_of((sid + t * _NS) * _ZR, 8)
            pltpu.sync_copy(qv[0].at[pl.ds(0, _ZR)], aggsh.at[pl.ds(r0, _ZR)])
            return carry

        lax.fori_loop(0, my_rchunks, zchunk, 0)
        plsc.subcore_barrier()

        def ebase(c):
            return pl.multiple_of((wid + c * nw) * _CH, 8)

        def issue_idx(c, b):
            pltpu.async_copy(src_hbm.at[pl.ds(ebase(c), _CH)], sidx[b],
                             semi[b])
            pltpu.async_copy(dst_hbm.at[pl.ds(ebase(c), _CH)], didx[b],
                             semi[b])

        def wait_idx(b):
            pltpu.make_async_copy(src_hbm.at[pl.ds(0, _CH)], sidx[b],
                                  semi[b]).wait()
            pltpu.make_async_copy(dst_hbm.at[pl.ds(0, _CH)], didx[b],
                                  semi[b]).wait()

        def issue_gather(b):
            pltpu.async_copy(qn_hbm.at[didx[b]], qv[b], semg[b])
            pltpu.async_copy(kn_hbm.at[sidx[b]], kv[b], semg[b])
            pltpu.async_copy(vn_hbm.at[sidx[b]], vv[b], semg[b])

        def wait_gather(b):
            pltpu.make_async_copy(qn_hbm.at[didx[b]], qv[b], semg[b]).wait()
            pltpu.make_async_copy(kn_hbm.at[sidx[b]], kv[b], semg[b]).wait()
            pltpu.make_async_copy(vn_hbm.at[sidx[b]], vv[b], semg[b]).wait()

        def compute(b):
            qb, kb, vb = qv[b], kv[b], vv[b]

            def edge_body(e, c2):
                acc = qb[e, pl.ds(0, 16)] * kb[e, pl.ds(0, 16)]
                for j in range(1, d // 16):
                    acc += qb[e, pl.ds(j * 16, 16)] * kb[e, pl.ds(j * 16, 16)]
                logit = jnp.sum(acc) * scale
                lv = jnp.zeros((16,), jnp.float32) + logit
                alpha = 1.0 / (1.0 + jnp.exp(-lv))
                for j in range(d // 16):
                    vb[e, pl.ds(j * 16, 16)] = alpha * vb[e, pl.ds(j * 16, 16)]
                return c2

            lax.fori_loop(0, _CH, edge_body, 0)

        for b in range(2):
            issue_idx(b, b)
            wait_idx(b)
            issue_gather(b)

        def pair_body(i, carry):
            for b in range(2):
                c = 2 * i + b
                wait_gather(b)
                compute(b)
                pltpu.sync_copy(vv[b], aggsh.at[didx[b]], add=True)

                @pl.when(c + 2 < chunks_per_w)
                def _():
                    issue_idx(c + 2, b)
                    wait_idx(b)
                    issue_gather(b)

            return carry

        lax.fori_loop(0, chunks_per_w // 2, pair_body, 0)
        plsc.subcore_barrier()

        def ochunk(t, carry):
            r0 = pl.multiple_of((sid + t * _NS) * _ZR, 8)
            pltpu.sync_copy(aggsh.at[pl.ds(r0, _ZR)], qv[0].at[pl.ds(0, _ZR)])
            pltpu.sync_copy(qv[0].at[pl.ds(0, _ZR)],
                            out_hbm.at[cid, pl.ds(r0, _ZR)])
            return carry

        lax.fori_loop(0, my_rchunks, ochunk, 0)

    return edge_kernel


def kernel(x, edge_index, inW, inb, bn_g, bn_b, qW, kW, vW, oW, ob, ln_g,
           ln_b, W1, b1, W2, b2):
    n, d = x.shape
    e = edge_index.shape[1]
    n_layers = qW.shape[0]
    grid = n // _BN

    blk = lambda: pl.BlockSpec((_BN, d), lambda i: (i, 0))
    blk2 = lambda: pl.BlockSpec((_BN, 2 * d), lambda i: (i, 0))
    row = lambda r=1: pl.BlockSpec((r, d), lambda i: (0, 0))
    wspec = lambda: pl.BlockSpec((d, d), lambda i: (0, 0))

    h_pre, s1, s2 = pl.pallas_call(
        _proj_stats_body,
        grid=(grid,),
        in_specs=[blk(), wspec(), row()],
        out_specs=[blk(), row(), row()],
        out_shape=[
            jax.ShapeDtypeStruct((n, d), jnp.float32),
            jax.ShapeDtypeStruct((1, d), jnp.float32),
            jax.ShapeDtypeStruct((1, d), jnp.float32),
        ],
    )(x, inW, inb.reshape(1, d))

    h, qn, kn, vn = pl.pallas_call(
        functools.partial(_norm_qkv_body, n_nodes=float(n)),
        grid=(grid,),
        in_specs=[blk(), row(), row(), row(), row(), wspec(), wspec(),
                  wspec()],
        out_specs=[blk(), blk(), blk(), blk()],
        out_shape=[jax.ShapeDtypeStruct((n, d), jnp.float32)] * 4,
    )(h_pre, s1, s2, bn_g.reshape(1, d), bn_b.reshape(1, d),
      qW[0], kW[0], vW[0])

    edge_fn = _edge_sc_kernel(n, e, d)
    pspec = pl.BlockSpec((_NC, _BN, d), lambda i: (0, i, 0))

    for i in range(n_layers):
        parts = edge_fn(qn, kn, vn, edge_index[0], edge_index[1])
        if i + 1 < n_layers:
            h, qn, kn, vn = pl.pallas_call(
                _post_qkv_body,
                grid=(grid,),
                in_specs=[blk(), pspec, wspec(), row(), row(), row(),
                          wspec(), wspec(), wspec()],
                out_specs=[blk(), blk(), blk(), blk()],
                out_shape=[jax.ShapeDtypeStruct((n, d), jnp.float32)] * 4,
            )(h, parts, oW[i], ob[i].reshape(1, d), ln_g[i].reshape(1, d),
              ln_b[i].reshape(1, d), qW[i + 1], kW[i + 1], vW[i + 1])
        else:
            s = pl.pallas_call(
                _post_pool_body,
                grid=(grid,),
                in_specs=[blk(), pspec, wspec(), row(), row(), row()],
                out_specs=row(),
                out_shape=jax.ShapeDtypeStruct((1, d), jnp.float32),
            )(h, parts, oW[i], ob[i].reshape(1, d), ln_g[i].reshape(1, d),
              ln_b[i].reshape(1, d))

    y = pl.pallas_call(
        functools.partial(_mlp_body, n_nodes=float(n)),
        grid=(1,),
        in_specs=[row(), wspec(), wspec(), row(),
                  pl.BlockSpec((d, 1), lambda i: (0, 0)),
                  pl.BlockSpec((1, 1), lambda i: (0, 0))],
        out_specs=pl.BlockSpec((1, 1), lambda i: (0, 0)),
        out_shape=jax.ShapeDtypeStruct((1, 1), jnp.float32),
    )(s, W1[:d], W1[d:], b1.reshape(1, d), W2, b2.reshape(1, 1))

    return y.reshape(())
